# baseline (speedup 1.0000x reference)
import functools

import jax
import jax.numpy as jnp
from jax import lax
from jax.experimental import pallas as pl
from jax.experimental.pallas import tpu as pltpu
from jax.experimental.pallas import tpu_sc as plsc

N_NODES = 10000
DEG = 32
D = 128
ALPHA = 0.2
LANES = 16

NSC = 2
NTILE = 16
N_PAD = 10240
NODES_SC = N_PAD // NSC
SLAB = D // NTILE
PC = 64
NCH = NODES_SC // PC
NT = NCH // 2
PAIRS_IT = 4
INNER = PC // (2 * PAIRS_IT)
RB = NODES_SC // NTILE


def _chain_sum(vals):
    chains = list(vals[:4])
    for j in range(4, len(vals)):
        chains[j % 4] = chains[j % 4] + vals[j]
    return (chains[0] + chains[1]) + (chains[2] + chains[3])


def _perm(x, patt):
    return lax.gather(
        x, patt.reshape(LANES, 1),
        lax.GatherDimensionNumbers(
            offset_dims=(), collapsed_slice_dims=(0,), start_index_map=(0,)),
        (1,), mode=lax.GatherScatterMode.PROMISE_IN_BOUNDS)


def _sc_aggregate(h, adj2d, scale):
    mesh = plsc.VectorSubcoreMesh(core_axis_name="c", subcore_axis_name="s")

    @functools.partial(
        pl.kernel,
        mesh=mesh,
        out_type=jax.ShapeDtypeStruct((N_PAD, D), jnp.float32),
        compiler_params=pltpu.CompilerParams(
            needs_layout_passes=False, use_tc_tiling_on_sc=False),
        scratch_types=[
            pltpu.VMEM((N_PAD, SLAB), jnp.float32),
            pltpu.VMEM((NODES_SC, SLAB), jnp.float32),
            pltpu.VMEM((PC * DEG,), jnp.int32),
            pltpu.VMEM((PC * DEG,), jnp.int32),
            pltpu.SemaphoreType.DMA,
            pltpu.SemaphoreType.DMA,
            pltpu.SemaphoreType.DMA,
        ],
    )
    def agg(h_hbm, adj_hbm, out_hbm, slab_v, out_v, adj0, adj1,
            sem0, sem1, semw):
        c = lax.axis_index("c")
        t = lax.axis_index("s")

        pltpu.sync_copy(
            h_hbm.at[pl.ds(0, N_NODES), pl.ds(t * SLAB, SLAB)],
            slab_v.at[pl.ds(0, N_NODES)])
        pltpu.async_copy(adj_hbm.at[c * NCH], adj0, sem0)

        iota = lax.iota(jnp.int32, LANES)
        coloff = jnp.bitwise_and(iota, SLAB - 1)
        hi8 = lax.shift_right_logical(iota, 3)
        patts = [hi8 + (2 * k) for k in range(DEG // 4)]
        rot8 = jnp.bitwise_xor(iota, 8)
        mask8 = iota < 8
        node0_sc = c * NODES_SC

        def node_sums(av0, av1):
            vals = []
            for av in (av0, av1):
                for k in range(DEG // 4):
                    rows = _perm(av, patts[k])
                    vals.append(plsc.load_gather(slab_v, [rows, coloff]))
            return _chain_sum(vals)

        def compute(g, adj_v):
            @plsc.parallel_loop(0, INNER, unroll=4)
            def inner(p0):
                for q in range(PAIRS_IT):
                    pair = p0 * PAIRS_IT + q
                    aoff = pair * 2 * DEG
                    av0 = adj_v[pl.ds(aoff, LANES)]
                    av1 = adj_v[pl.ds(aoff + LANES, LANES)]
                    bv0 = adj_v[pl.ds(aoff + 2 * LANES, LANES)]
                    bv1 = adj_v[pl.ds(aoff + 3 * LANES, LANES)]
                    acc_i = node_sums(av0, av1)
                    acc_j = node_sums(bv0, bv1)
                    u = jnp.where(mask8, acc_i, acc_j)
                    v = jnp.where(mask8, _perm(acc_i, rot8),
                                  _perm(acc_j, rot8))
                    tot = u + v
                    nloc = g * PC + pair * 2
                    rows_l = jnp.full((LANES,), nloc, jnp.int32) + hi8
                    slf = plsc.load_gather(slab_v, [rows_l + node0_sc, coloff])
                    plsc.store_scatter(out_v, [rows_l, coloff],
                                       (tot + slf) * scale)

            pltpu.async_copy(
                out_v.at[pl.ds(g * PC, PC)],
                out_hbm.at[pl.ds(node0_sc + g * PC, PC),
                           pl.ds(t * SLAB, SLAB)],
                semw)

        def body(it, carry):
            g0 = 2 * it
            pltpu.async_copy(adj_hbm.at[c * NCH + g0 + 1], adj1, sem1)
            pltpu.make_async_copy(adj_hbm.at[0], adj0, sem0).wait()
            compute(g0, adj0)
            pltpu.async_copy(adj_hbm.at[c * NCH + g0 + 2], adj0, sem0)
            pltpu.make_async_copy(adj_hbm.at[0], adj1, sem1).wait()
            compute(g0 + 1, adj1)
            return carry

        lax.fori_loop(0, NT, body, 0)
        pltpu.make_async_copy(adj_hbm.at[0], adj0, sem0).wait()

        def drain(g, carry):
            pltpu.make_async_copy(
                out_v.at[pl.ds(0, PC)],
                out_hbm.at[pl.ds(node0_sc, PC), pl.ds(t * SLAB, SLAB)],
                semw).wait()
            return carry

        lax.fori_loop(0, NCH, drain, 0)

    return agg(h, adj2d)


def _tc_mlp(feat, w):
    blk = 512

    def body(f_ref, w_ref, o_ref):
        x = jnp.dot(f_ref[...], w_ref[...], preferred_element_type=jnp.float32)
        x = jnp.where(x >= 0, x, ALPHA * x)
        nrm = jnp.sqrt(jnp.sum(x * x, axis=1, keepdims=True))
        o_ref[...] = x / jnp.maximum(nrm, 1e-12)

    return pl.pallas_call(
        body,
        grid=(N_PAD // blk,),
        in_specs=[
            pl.BlockSpec((blk, D), lambda i: (i, 0)),
            pl.BlockSpec((D, D), lambda i: (0, 0)),
        ],
        out_specs=pl.BlockSpec((blk, D), lambda i: (i, 0)),
        out_shape=jax.ShapeDtypeStruct((N_PAD, D), jnp.float32),
    )(feat, w)


def kernel(h, adj, aggregate_num, W_gcn):
    del aggregate_num
    h = h.astype(jnp.float32)
    adj32 = adj.astype(jnp.int32)
    scale = 1.0 / (adj.shape[1] + 1)
    adj_pad = jnp.zeros((N_PAD, DEG), jnp.int32).at[:N_NODES].set(adj32)
    adj2d = jnp.zeros((NSC * NCH + 1, PC * DEG), jnp.int32)
    adj2d = adj2d.at[:NSC * NCH].set(adj_pad.reshape(NSC * NCH, PC * DEG))
    feat = _sc_aggregate(h, adj2d, scale)
    out = _tc_mlp(feat, W_gcn)
    return out[:N_NODES]


# speedup vs baseline: 1.2024x; 1.2024x over previous
import functools

import jax
import jax.numpy as jnp
from jax import lax
from jax.experimental import pallas as pl
from jax.experimental.pallas import tpu as pltpu
from jax.experimental.pallas import tpu_sc as plsc

N_NODES = 10000
DEG = 32
D = 128
ALPHA = 0.2
LANES = 16

NSC = 2
NTILE = 16
N_PAD = 10240
NODES_SC = N_PAD // NSC
SLAB = D // NTILE
PC = 64
NCH = NODES_SC // PC
NT = NCH // 2
PAIRS_IT = 8
INNER = PC // (2 * PAIRS_IT)
RB = NODES_SC // NTILE


def _chain_sum(vals):
    chains = list(vals[:4])
    for j in range(4, len(vals)):
        chains[j % 4] = chains[j % 4] + vals[j]
    return (chains[0] + chains[1]) + (chains[2] + chains[3])


def _perm(x, patt):
    return lax.gather(
        x, patt.reshape(LANES, 1),
        lax.GatherDimensionNumbers(
            offset_dims=(), collapsed_slice_dims=(0,), start_index_map=(0,)),
        (1,), mode=lax.GatherScatterMode.PROMISE_IN_BOUNDS)


def _sc_aggregate(h, adj2d, scale):
    mesh = plsc.VectorSubcoreMesh(core_axis_name="c", subcore_axis_name="s")

    @functools.partial(
        pl.kernel,
        mesh=mesh,
        out_type=jax.ShapeDtypeStruct((N_PAD, D), jnp.float32),
        compiler_params=pltpu.CompilerParams(
            needs_layout_passes=False, use_tc_tiling_on_sc=False),
        scratch_types=[
            pltpu.VMEM((N_PAD, SLAB), jnp.float32),
            pltpu.VMEM((NODES_SC, SLAB), jnp.float32),
            pltpu.VMEM((PC * DEG,), jnp.int32),
            pltpu.VMEM((PC * DEG,), jnp.int32),
            pltpu.SemaphoreType.DMA,
            pltpu.SemaphoreType.DMA,
            pltpu.SemaphoreType.DMA,
        ],
    )
    def agg(h_hbm, adj_hbm, out_hbm, slab_v, out_v, adj0, adj1,
            sem0, sem1, semw):
        c = lax.axis_index("c")
        t = lax.axis_index("s")

        pltpu.sync_copy(
            h_hbm.at[pl.ds(0, N_NODES), pl.ds(t * SLAB, SLAB)],
            slab_v.at[pl.ds(0, N_NODES)])
        pltpu.async_copy(adj_hbm.at[c * NCH], adj0, sem0)

        iota = lax.iota(jnp.int32, LANES)
        coloff = jnp.bitwise_and(iota, SLAB - 1)
        hi8 = lax.shift_right_logical(iota, 3)
        patts = [hi8 + (2 * k) for k in range(DEG // 4)]
        rot8 = jnp.bitwise_xor(iota, 8)
        mask8 = iota < 8
        node0_sc = c * NODES_SC

        def node_sums(av0, av1):
            vals = []
            for av in (av0, av1):
                for k in range(DEG // 4):
                    rows = _perm(av, patts[k])
                    vals.append(plsc.load_gather(slab_v, [rows, coloff]))
            return _chain_sum(vals)

        def compute(g, adj_v):
            def inner(p0, carry):
                for q in range(PAIRS_IT):
                    pair = p0 * PAIRS_IT + q
                    aoff = pair * 2 * DEG
                    av0 = adj_v[pl.ds(aoff, LANES)]
                    av1 = adj_v[pl.ds(aoff + LANES, LANES)]
                    bv0 = adj_v[pl.ds(aoff + 2 * LANES, LANES)]
                    bv1 = adj_v[pl.ds(aoff + 3 * LANES, LANES)]
                    acc_i = node_sums(av0, av1)
                    acc_j = node_sums(bv0, bv1)
                    u = jnp.where(mask8, acc_i, acc_j)
                    v = jnp.where(mask8, _perm(acc_i, rot8),
                                  _perm(acc_j, rot8))
                    tot = u + v
                    nloc = g * PC + pair * 2
                    rows_l = jnp.full((LANES,), nloc, jnp.int32) + hi8
                    slf = plsc.load_gather(slab_v, [rows_l + node0_sc, coloff])
                    plsc.store_scatter(out_v, [rows_l, coloff],
                                       (tot + slf) * scale)
                return carry

            lax.fori_loop(0, INNER, inner, 0)
            pltpu.async_copy(
                out_v.at[pl.ds(g * PC, PC)],
                out_hbm.at[pl.ds(node0_sc + g * PC, PC),
                           pl.ds(t * SLAB, SLAB)],
                semw)

        def body(it, carry):
            g0 = 2 * it
            pltpu.async_copy(adj_hbm.at[c * NCH + g0 + 1], adj1, sem1)
            pltpu.make_async_copy(adj_hbm.at[0], adj0, sem0).wait()
            compute(g0, adj0)
            pltpu.async_copy(adj_hbm.at[c * NCH + g0 + 2], adj0, sem0)
            pltpu.make_async_copy(adj_hbm.at[0], adj1, sem1).wait()
            compute(g0 + 1, adj1)
            return carry

        lax.fori_loop(0, NT, body, 0)
        pltpu.make_async_copy(adj_hbm.at[0], adj0, sem0).wait()

        def drain(g, carry):
            pltpu.make_async_copy(
                out_v.at[pl.ds(0, PC)],
                out_hbm.at[pl.ds(node0_sc, PC), pl.ds(t * SLAB, SLAB)],
                semw).wait()
            return carry

        lax.fori_loop(0, NCH, drain, 0)

    return agg(h, adj2d)


def _tc_mlp(feat, w):
    blk = 400

    def body(f_ref, w_ref, o_ref):
        x = jnp.dot(f_ref[...], w_ref[...], preferred_element_type=jnp.float32)
        x = jnp.where(x >= 0, x, ALPHA * x)
        nrm = jnp.sqrt(jnp.sum(x * x, axis=1, keepdims=True))
        o_ref[...] = x / jnp.maximum(nrm, 1e-12)

    return pl.pallas_call(
        body,
        grid=(N_NODES // blk,),
        in_specs=[
            pl.BlockSpec((blk, D), lambda i: (i, 0)),
            pl.BlockSpec((D, D), lambda i: (0, 0)),
        ],
        out_specs=pl.BlockSpec((blk, D), lambda i: (i, 0)),
        out_shape=jax.ShapeDtypeStruct((N_NODES, D), jnp.float32),
    )(feat, w)


def kernel(h, adj, aggregate_num, W_gcn):
    del aggregate_num
    h = h.astype(jnp.float32)
    adj32 = adj.astype(jnp.int32)
    scale = 1.0 / (adj.shape[1] + 1)
    adj_pad = jnp.zeros((N_PAD, DEG), jnp.int32).at[:N_NODES].set(adj32)
    adj2d = jnp.zeros((NSC * NCH + 1, PC * DEG), jnp.int32)
    adj2d = adj2d.at[:NSC * NCH].set(adj_pad.reshape(NSC * NCH, PC * DEG))
    feat = _sc_aggregate(h, adj2d, scale)
    return _tc_mlp(feat, W_gcn)


# speedup vs baseline: 1.2353x; 1.0274x over previous
import functools

import jax
import jax.numpy as jnp
from jax import lax
from jax.experimental import pallas as pl
from jax.experimental.pallas import tpu as pltpu
from jax.experimental.pallas import tpu_sc as plsc

N_NODES = 10000
DEG = 32
D = 128
ALPHA = 0.2
LANES = 16

NSC = 2
NTILE = 16
N_PAD = 10240
NODES_SC = N_PAD // NSC
SLAB = D // NTILE
PC = 80
NCH = NODES_SC // PC
NT = NCH // 2
PAIRS_IT = 8
INNER = PC // (2 * PAIRS_IT)
RB = NODES_SC // NTILE


def _chain_sum(vals):
    chains = list(vals[:4])
    for j in range(4, len(vals)):
        chains[j % 4] = chains[j % 4] + vals[j]
    return (chains[0] + chains[1]) + (chains[2] + chains[3])


def _perm(x, patt):
    return lax.gather(
        x, patt.reshape(LANES, 1),
        lax.GatherDimensionNumbers(
            offset_dims=(), collapsed_slice_dims=(0,), start_index_map=(0,)),
        (1,), mode=lax.GatherScatterMode.PROMISE_IN_BOUNDS)


def _sc_aggregate(h, adj2d, scale):
    mesh = plsc.VectorSubcoreMesh(core_axis_name="c", subcore_axis_name="s")

    @functools.partial(
        pl.kernel,
        mesh=mesh,
        out_type=jax.ShapeDtypeStruct((N_PAD, D), jnp.float32),
        compiler_params=pltpu.CompilerParams(
            needs_layout_passes=False, use_tc_tiling_on_sc=False),
        scratch_types=[
            pltpu.VMEM((N_PAD, SLAB), jnp.float32),
            pltpu.VMEM((NODES_SC, SLAB), jnp.float32),
            pltpu.VMEM((PC * DEG,), jnp.int32),
            pltpu.VMEM((PC * DEG,), jnp.int32),
            pltpu.SemaphoreType.DMA,
            pltpu.SemaphoreType.DMA,
            pltpu.SemaphoreType.DMA,
        ],
    )
    def agg(h_hbm, adj_hbm, out_hbm, slab_v, out_v, adj0, adj1,
            sem0, sem1, semw):
        c = lax.axis_index("c")
        t = lax.axis_index("s")

        pltpu.sync_copy(
            h_hbm.at[pl.ds(0, N_NODES), pl.ds(t * SLAB, SLAB)],
            slab_v.at[pl.ds(0, N_NODES)])
        pltpu.async_copy(adj_hbm.at[c * NCH], adj0, sem0)

        iota = lax.iota(jnp.int32, LANES)
        coloff = jnp.bitwise_and(iota, SLAB - 1)
        hi8 = lax.shift_right_logical(iota, 3)
        patts = [hi8 + (2 * k) for k in range(DEG // 4)]
        rot8 = jnp.bitwise_xor(iota, 8)
        mask8 = iota < 8
        node0_sc = c * NODES_SC

        def node_sums(av0, av1):
            vals = []
            for av in (av0, av1):
                for k in range(DEG // 4):
                    rows = _perm(av, patts[k])
                    vals.append(plsc.load_gather(slab_v, [rows, coloff]))
            return _chain_sum(vals)

        def compute(g, adj_v):
            def inner(p0, carry):
                for q in range(PAIRS_IT):
                    pair = p0 * PAIRS_IT + q
                    aoff = pair * 2 * DEG
                    av0 = adj_v[pl.ds(aoff, LANES)]
                    av1 = adj_v[pl.ds(aoff + LANES, LANES)]
                    bv0 = adj_v[pl.ds(aoff + 2 * LANES, LANES)]
                    bv1 = adj_v[pl.ds(aoff + 3 * LANES, LANES)]
                    acc_i = node_sums(av0, av1)
                    acc_j = node_sums(bv0, bv1)
                    u = jnp.where(mask8, acc_i, acc_j)
                    v = jnp.where(mask8, _perm(acc_i, rot8),
                                  _perm(acc_j, rot8))
                    tot = u + v
                    nloc = g * PC + pair * 2
                    rows_l = jnp.full((LANES,), nloc, jnp.int32) + hi8
                    slf = plsc.load_gather(slab_v, [rows_l + node0_sc, coloff])
                    plsc.store_scatter(out_v, [rows_l, coloff],
                                       (tot + slf) * scale)
                return carry

            lax.fori_loop(0, INNER, inner, 0)
            pltpu.async_copy(
                out_v.at[pl.ds(g * PC, PC)],
                out_hbm.at[pl.ds(node0_sc + g * PC, PC),
                           pl.ds(t * SLAB, SLAB)],
                semw)

        def body(it, carry):
            g0 = 2 * it
            pltpu.async_copy(adj_hbm.at[c * NCH + g0 + 1], adj1, sem1)
            pltpu.make_async_copy(adj_hbm.at[0], adj0, sem0).wait()
            compute(g0, adj0)
            pltpu.async_copy(adj_hbm.at[c * NCH + g0 + 2], adj0, sem0)
            pltpu.make_async_copy(adj_hbm.at[0], adj1, sem1).wait()
            compute(g0 + 1, adj1)
            return carry

        lax.fori_loop(0, NT, body, 0)
        pltpu.make_async_copy(adj_hbm.at[0], adj0, sem0).wait()

        def drain(g, carry):
            pltpu.make_async_copy(
                out_v.at[pl.ds(0, PC)],
                out_hbm.at[pl.ds(node0_sc, PC), pl.ds(t * SLAB, SLAB)],
                semw).wait()
            return carry

        lax.fori_loop(0, NCH, drain, 0)

    return agg(h, adj2d)


def _tc_mlp(feat, w):
    blk = 400

    def body(f_ref, w_ref, o_ref):
        x = jnp.dot(f_ref[...], w_ref[...], preferred_element_type=jnp.float32)
        x = jnp.where(x >= 0, x, ALPHA * x)
        nrm = jnp.sqrt(jnp.sum(x * x, axis=1, keepdims=True))
        o_ref[...] = x / jnp.maximum(nrm, 1e-12)

    return pl.pallas_call(
        body,
        grid=(N_NODES // blk,),
        in_specs=[
            pl.BlockSpec((blk, D), lambda i: (i, 0)),
            pl.BlockSpec((D, D), lambda i: (0, 0)),
        ],
        out_specs=pl.BlockSpec((blk, D), lambda i: (i, 0)),
        out_shape=jax.ShapeDtypeStruct((N_NODES, D), jnp.float32),
    )(feat, w)


def kernel(h, adj, aggregate_num, W_gcn):
    del aggregate_num
    h = h.astype(jnp.float32)
    adj32 = adj.astype(jnp.int32)
    scale = 1.0 / (adj.shape[1] + 1)
    adj2d = jnp.zeros((NSC * NCH + 1, PC * DEG), jnp.int32)
    adj2d = adj2d.at[:N_NODES * DEG // (PC * DEG)].set(
        adj32.reshape(N_NODES * DEG // (PC * DEG), PC * DEG))
    feat = _sc_aggregate(h, adj2d, scale)
    return _tc_mlp(feat, W_gcn)
